# baseline (speedup 1.0000x reference)
import jax
import jax.numpy as jnp
from jax.experimental import pallas as pl

TOP_K = 8
THRESHOLD = 1e-4
INV_TEMPERATURE = 10.0
CHUNK = 8


def _graph_block_kernel(x_ref, o_ref):
    Tt = x_ref.shape[2]
    for c in range(Tt // CHUNK):
        x = jnp.transpose(
            x_ref[0, :, c * CHUNK:(c + 1) * CHUNK, :], (1, 0, 2)
        )
        norm2 = jnp.sum(x * x, axis=-1, keepdims=True)
        xn = x * jax.lax.rsqrt(jnp.maximum(norm2, 1e-24))
        adj = jax.lax.dot_general(
            xn, xn, (((2,), (2,)), ((0,), (0,))),
            preferred_element_type=jnp.float32,
        )

        e = jnp.exp((adj - 1.0) * INV_TEMPERATURE)
        s = jnp.sum(e, axis=-1, keepdims=True)
        r = 1.0 / s

        row = jax.lax.broadcasted_iota(jnp.int32, e.shape, 1)
        col = jax.lax.broadcasted_iota(jnp.int32, e.shape, 2)
        work = jnp.where(row == col, -1.0, e)
        for _ in range(TOP_K - 2):
            mx = jnp.max(work, axis=-1, keepdims=True)
            work = jnp.where(work < mx, work, -1.0)
        t8 = jnp.max(work, axis=-1, keepdims=True)

        keep = (e >= t8) & (e > THRESHOLD * s)
        a = jnp.where(keep, e, 0.0) * r
        o_ref[0, c * CHUNK:(c + 1) * CHUNK] = (
            (a + jnp.transpose(a, (0, 2, 1))) * 0.5
        )


def kernel(features):
    B, N, T, D = features.shape
    Tt = 64
    return pl.pallas_call(
        _graph_block_kernel,
        grid=(B, T // Tt),
        in_specs=[pl.BlockSpec((1, N, Tt, D), lambda b, t: (b, 0, t, 0))],
        out_specs=pl.BlockSpec((1, Tt, N, N), lambda b, t: (b, t, 0, 0)),
        out_shape=jax.ShapeDtypeStruct((B, T, N, N), jnp.float32),
    )(features)


# speedup vs baseline: 1.1208x; 1.1208x over previous
import jax
import jax.numpy as jnp
from jax.experimental import pallas as pl
from jax.experimental.pallas import tpu as pltpu

TOP_K = 8
THRESHOLD = 1e-4
INV_TEMPERATURE = 10.0
CHUNK = 64


def _graph_block_kernel(x_ref, o_ref):
    Tt = x_ref.shape[2]
    for c in range(Tt // CHUNK):
        x = jnp.transpose(
            x_ref[0, :, c * CHUNK:(c + 1) * CHUNK, :], (1, 0, 2)
        )
        norm2 = jnp.sum(x * x, axis=-1, keepdims=True)
        xn = x * jax.lax.rsqrt(jnp.maximum(norm2, 1e-24))
        adj = jax.lax.dot_general(
            xn, xn, (((2,), (2,)), ((0,), (0,))),
            preferred_element_type=jnp.float32,
        )

        e = jnp.exp((adj - 1.0) * INV_TEMPERATURE)
        s = jnp.sum(e, axis=-1, keepdims=True)
        r = 1.0 / s

        row = jax.lax.broadcasted_iota(jnp.int32, e.shape, 1)
        col = jax.lax.broadcasted_iota(jnp.int32, e.shape, 2)
        work = jnp.where(row == col, -1.0, e)
        for _ in range(TOP_K - 2):
            mx = jnp.max(work, axis=-1, keepdims=True)
            work = jnp.where(work < mx, work, -1.0)
        t8 = jnp.max(work, axis=-1, keepdims=True)

        thr = jnp.maximum(t8, THRESHOLD * s)
        a = jnp.where(e >= thr, e, 0.0) * r
        o_ref[0, c * CHUNK:(c + 1) * CHUNK] = (
            (a + jnp.transpose(a, (0, 2, 1))) * 0.5
        )


def kernel(features):
    B, N, T, D = features.shape
    Tt = 64
    return pl.pallas_call(
        _graph_block_kernel,
        grid=(B, T // Tt),
        in_specs=[pl.BlockSpec((1, N, Tt, D), lambda b, t: (b, 0, t, 0))],
        out_specs=pl.BlockSpec((1, Tt, N, N), lambda b, t: (b, t, 0, 0)),
        out_shape=jax.ShapeDtypeStruct((B, T, N, N), jnp.float32),
        compiler_params=pltpu.CompilerParams(
            dimension_semantics=("parallel", "parallel"),
        ),
    )(features)


# speedup vs baseline: 1.2071x; 1.0770x over previous
import jax
import jax.numpy as jnp
from jax.experimental import pallas as pl
from jax.experimental.pallas import tpu as pltpu

TOP_K = 8
THRESHOLD = 1e-4
INV_TEMPERATURE = 10.0
CHUNK = 64


def _graph_block_kernel(x_ref, o_ref):
    Tt = x_ref.shape[2]
    for c in range(Tt // CHUNK):
        x = jnp.transpose(
            x_ref[0, :, c * CHUNK:(c + 1) * CHUNK, :], (1, 0, 2)
        )
        norm2 = jnp.sum(x * x, axis=-1, keepdims=True)
        xn = x * jax.lax.rsqrt(jnp.maximum(norm2, 1e-24))
        adj = jax.lax.dot_general(
            xn, xn, (((2,), (2,)), ((0,), (0,))),
            preferred_element_type=jnp.float32,
        )

        e = jnp.exp((adj - 1.0) * INV_TEMPERATURE)
        s = jnp.sum(e, axis=-1, keepdims=True)
        rh = 0.5 / s

        row = jax.lax.broadcasted_iota(jnp.int32, e.shape, 1)
        col = jax.lax.broadcasted_iota(jnp.int32, e.shape, 2)
        work = jnp.where(row == col, -1.0, e)
        for _ in range(TOP_K - 2):
            mx = jnp.max(work, axis=1, keepdims=True)
            work = jnp.where(work < mx, work, -1.0)
        t8c = jnp.max(work, axis=1, keepdims=True)
        t8 = jnp.transpose(t8c, (0, 2, 1))

        thr = jnp.maximum(t8, THRESHOLD * s)
        thr_c = jnp.transpose(thr, (0, 2, 1))
        rh_c = jnp.transpose(rh, (0, 2, 1))
        p = jnp.where(e >= thr, e, 0.0) * rh
        q = jnp.where(e >= thr_c, e, 0.0) * rh_c
        o_ref[0, c * CHUNK:(c + 1) * CHUNK] = p + q


def kernel(features):
    B, N, T, D = features.shape
    Tt = 64
    return pl.pallas_call(
        _graph_block_kernel,
        grid=(B, T // Tt),
        in_specs=[pl.BlockSpec((1, N, Tt, D), lambda b, t: (b, 0, t, 0))],
        out_specs=pl.BlockSpec((1, Tt, N, N), lambda b, t: (b, t, 0, 0)),
        out_shape=jax.ShapeDtypeStruct((B, T, N, N), jnp.float32),
        compiler_params=pltpu.CompilerParams(
            dimension_semantics=("parallel", "parallel"),
        ),
    )(features)


# speedup vs baseline: 1.2712x; 1.0531x over previous
import jax
import jax.numpy as jnp
from jax.experimental import pallas as pl
from jax.experimental.pallas import tpu as pltpu

TOP_K = 8
THRESHOLD = 1e-4
INV_TEMPERATURE = 10.0
CHUNK = 64


def _graph_block_kernel(x_ref, o_ref):
    Tt = x_ref.shape[2]
    for c in range(Tt // CHUNK):
        x = jnp.transpose(
            x_ref[0, :, c * CHUNK:(c + 1) * CHUNK, :], (1, 0, 2)
        )
        norm2 = jnp.sum(x * x, axis=-1, keepdims=True)
        xn = x * jax.lax.rsqrt(jnp.maximum(norm2, 1e-24))
        adj = jax.lax.dot_general(
            xn, xn, (((2,), (2,)), ((0,), (0,))),
            preferred_element_type=jnp.float32,
        )

        e = jnp.exp((adj - 1.0) * INV_TEMPERATURE)
        s = jnp.sum(e, axis=-1, keepdims=True)
        rh = 0.5 / s

        row = jax.lax.broadcasted_iota(jnp.int32, e.shape, 1)
        col = jax.lax.broadcasted_iota(jnp.int32, e.shape, 2)
        work = jnp.where(row == col, -1.0, e)
        for _ in range(TOP_K - 2):
            mx = jnp.max(work, axis=1, keepdims=True)
            work = jnp.where(work < mx, work, -1.0)
        t8c = jnp.max(work, axis=1, keepdims=True)
        t8 = jnp.transpose(t8c, (0, 2, 1))

        thr = jnp.maximum(t8, THRESHOLD * s)
        both = jnp.concatenate((thr, rh), axis=2)
        both_c = jnp.transpose(both, (0, 2, 1))
        thr_c = both_c[:, 0:1, :]
        rh_c = both_c[:, 1:2, :]
        p = jnp.where(e >= thr, e, 0.0) * rh
        q = jnp.where(e >= thr_c, e, 0.0) * rh_c
        o_ref[0, c * CHUNK:(c + 1) * CHUNK] = p + q


def kernel(features):
    B, N, T, D = features.shape
    Tt = 64
    return pl.pallas_call(
        _graph_block_kernel,
        grid=(B, T // Tt),
        in_specs=[pl.BlockSpec((1, N, Tt, D), lambda b, t: (b, 0, t, 0))],
        out_specs=pl.BlockSpec((1, Tt, N, N), lambda b, t: (b, t, 0, 0)),
        out_shape=jax.ShapeDtypeStruct((B, T, N, N), jnp.float32),
        compiler_params=pltpu.CompilerParams(
            dimension_semantics=("parallel", "parallel"),
        ),
    )(features)


# speedup vs baseline: 1.4439x; 1.1359x over previous
import jax
import jax.numpy as jnp
from jax.experimental import pallas as pl
from jax.experimental.pallas import tpu as pltpu

TOP_K = 8
THRESHOLD = 1e-4
INV_TEMPERATURE = 10.0

_SORT8 = (
    (0, 1), (2, 3), (4, 5), (6, 7),
    (0, 2), (1, 3), (4, 6), (5, 7),
    (1, 2), (5, 6),
    (0, 4), (1, 5), (2, 6), (3, 7),
    (2, 4), (3, 5),
    (1, 2), (3, 4), (5, 6),
)
_CLEAN8 = (
    (0, 4), (1, 5), (2, 6), (3, 7),
    (0, 2), (1, 3), (4, 6), (5, 7),
    (0, 1), (2, 3), (4, 5), (6, 7),
)


def _cmpex(s, pairs):
    for i, j in pairs:
        lo = jnp.minimum(s[i], s[j])
        s[j] = jnp.maximum(s[i], s[j])
        s[i] = lo
    return s


def _row_top8_threshold(e):
    n = e.shape[1]
    g = n // 8
    s = [e[:, i * g:(i + 1) * g, :] for i in range(8)]
    s = _cmpex(s, _SORT8)
    for d in (4, 2):
        r = [jnp.roll(x, -d, axis=1) for x in s]
        s = [jnp.maximum(s[i], r[7 - i]) for i in range(8)]
        s = _cmpex(s, _CLEAN8)
    r = [jnp.roll(x, -1, axis=1) for x in s]
    s = [jnp.maximum(s[i], r[7 - i]) for i in range(8)]
    t = jnp.minimum(jnp.minimum(jnp.minimum(s[0], s[1]),
                                jnp.minimum(s[2], s[3])),
                    jnp.minimum(jnp.minimum(s[4], s[5]),
                                jnp.minimum(s[6], s[7])))
    return t[:, 0:1, :]


def _graph_block_kernel(x_ref, o_ref):
    x = jnp.transpose(x_ref[0], (1, 0, 2))
    norm2 = jnp.sum(x * x, axis=-1, keepdims=True)
    xn = x * jax.lax.rsqrt(jnp.maximum(norm2, 1e-24))
    adj = jax.lax.dot_general(
        xn, xn, (((2,), (2,)), ((0,), (0,))),
        preferred_element_type=jnp.float32,
    )

    e = jnp.exp((adj - 1.0) * INV_TEMPERATURE)
    s = jnp.sum(e, axis=-1, keepdims=True)
    rh = 0.5 / s

    t8c = _row_top8_threshold(e)
    t8 = jnp.transpose(t8c, (0, 2, 1))

    thr = jnp.maximum(t8, THRESHOLD * s)
    both = jnp.concatenate((thr, rh), axis=2)
    both_c = jnp.transpose(both, (0, 2, 1))
    thr_c = both_c[:, 0:1, :]
    rh_c = both_c[:, 1:2, :]
    p = jnp.where(e >= thr, e, 0.0) * rh
    q = jnp.where(e >= thr_c, e, 0.0) * rh_c
    o_ref[0] = p + q


def kernel(features):
    B, N, T, D = features.shape
    Tt = 64
    return pl.pallas_call(
        _graph_block_kernel,
        grid=(B, T // Tt),
        in_specs=[pl.BlockSpec((1, N, Tt, D), lambda b, t: (b, 0, t, 0))],
        out_specs=pl.BlockSpec((1, Tt, N, N), lambda b, t: (b, t, 0, 0)),
        out_shape=jax.ShapeDtypeStruct((B, T, N, N), jnp.float32),
        compiler_params=pltpu.CompilerParams(
            dimension_semantics=("parallel", "parallel"),
        ),
    )(features)


# speedup vs baseline: 1.4748x; 1.0214x over previous
import jax
import jax.numpy as jnp
from jax.experimental import pallas as pl
from jax.experimental.pallas import tpu as pltpu

TOP_K = 8
THRESHOLD = 1e-4
INV_TEMPERATURE = 10.0

_SORT8 = (
    (0, 1), (2, 3), (4, 5), (6, 7),
    (0, 2), (1, 3), (4, 6), (5, 7),
    (1, 2), (5, 6),
    (0, 4), (1, 5), (2, 6), (3, 7),
    (2, 4), (3, 5),
    (1, 2), (3, 4), (5, 6),
)
_CLEAN8 = (
    (0, 4), (1, 5), (2, 6), (3, 7),
    (0, 2), (1, 3), (4, 6), (5, 7),
    (0, 1), (2, 3), (4, 5), (6, 7),
)


def _cmpex(s, pairs):
    for i, j in pairs:
        lo = jnp.minimum(s[i], s[j])
        s[j] = jnp.maximum(s[i], s[j])
        s[i] = lo
    return s


def _row_top8_threshold(e):
    n = e.shape[1]
    g = n // 8
    s = [e[:, i * g:(i + 1) * g, :] for i in range(8)]
    s = _cmpex(s, _SORT8)
    for d in (4, 2):
        r = [jnp.roll(x, -d, axis=1) for x in s]
        s = [jnp.maximum(s[i], r[7 - i]) for i in range(8)]
        s = _cmpex(s, _CLEAN8)
    r = [jnp.roll(x, -1, axis=1) for x in s]
    s = [jnp.maximum(s[i], r[7 - i]) for i in range(8)]
    t = jnp.minimum(jnp.minimum(jnp.minimum(s[0], s[1]),
                                jnp.minimum(s[2], s[3])),
                    jnp.minimum(jnp.minimum(s[4], s[5]),
                                jnp.minimum(s[6], s[7])))
    return t[:, 0:1, :]


def _graph_block_kernel(x_ref, o_ref):
    x = jnp.transpose(x_ref[0], (1, 0, 2))
    norm2 = jnp.sum(x * x, axis=-1, keepdims=True)
    xn = x * jax.lax.rsqrt(jnp.maximum(norm2, 1e-24))
    adj = jax.lax.dot_general(
        xn, xn, (((2,), (2,)), ((0,), (0,))),
        preferred_element_type=jnp.float32,
    )

    e = jnp.exp((adj - 1.0) * INV_TEMPERATURE)
    s = jnp.sum(e, axis=-1, keepdims=True)
    rh = 0.5 / s

    t8c = _row_top8_threshold(e)
    t8 = jnp.transpose(t8c, (0, 2, 1))

    thr = jnp.maximum(t8, THRESHOLD * s)
    both = jnp.concatenate((thr, rh), axis=2)
    both_c = jnp.transpose(both, (0, 2, 1))
    thr_c = both_c[:, 0:1, :]
    rh_c = both_c[:, 1:2, :]
    p = jnp.where(e >= thr, e, 0.0) * rh
    q = jnp.where(e >= thr_c, e, 0.0) * rh_c
    o_ref[0] = p + q


def kernel(features):
    B, N, T, D = features.shape
    Tt = 128
    return pl.pallas_call(
        _graph_block_kernel,
        grid=(B, T // Tt),
        in_specs=[pl.BlockSpec((1, N, Tt, D), lambda b, t: (b, 0, t, 0))],
        out_specs=pl.BlockSpec((1, Tt, N, N), lambda b, t: (b, t, 0, 0)),
        out_shape=jax.ShapeDtypeStruct((B, T, N, N), jnp.float32),
        compiler_params=pltpu.CompilerParams(
            dimension_semantics=("parallel", "parallel"),
        ),
    )(features)


# speedup vs baseline: 1.4920x; 1.0116x over previous
import jax
import jax.numpy as jnp
from jax.experimental import pallas as pl
from jax.experimental.pallas import tpu as pltpu

TOP_K = 8
THRESHOLD = 1e-4
INV_TEMPERATURE = 10.0

_SORT8 = (
    (0, 1), (2, 3), (4, 5), (6, 7),
    (0, 2), (1, 3), (4, 6), (5, 7),
    (1, 2), (5, 6),
    (0, 4), (1, 5), (2, 6), (3, 7),
    (2, 4), (3, 5),
    (1, 2), (3, 4), (5, 6),
)
_CLEAN8 = (
    (0, 4), (1, 5), (2, 6), (3, 7),
    (0, 2), (1, 3), (4, 6), (5, 7),
    (0, 1), (2, 3), (4, 5), (6, 7),
)


def _cmpex(s, pairs):
    for i, j in pairs:
        lo = jnp.minimum(s[i], s[j])
        s[j] = jnp.maximum(s[i], s[j])
        s[i] = lo
    return s


def _row_top8_threshold(e):
    n = e.shape[1]
    g = n // 8
    s = [e[:, i * g:(i + 1) * g, :] for i in range(8)]
    s = _cmpex(s, _SORT8)
    for d in (4, 2):
        r = [jnp.roll(x, -d, axis=1) for x in s]
        s = [jnp.maximum(s[i], r[7 - i]) for i in range(8)]
        s = _cmpex(s, _CLEAN8)
    r = [jnp.roll(x, -1, axis=1) for x in s]
    s = [jnp.maximum(s[i], r[7 - i]) for i in range(8)]
    t = jnp.minimum(jnp.minimum(jnp.minimum(s[0], s[1]),
                                jnp.minimum(s[2], s[3])),
                    jnp.minimum(jnp.minimum(s[4], s[5]),
                                jnp.minimum(s[6], s[7])))
    return t[:, 0:1, :]


def _graph_block_kernel(x_ref, o_ref):
    x = jnp.transpose(x_ref[0], (1, 0, 2))
    norm2 = jnp.sum(x * x, axis=-1, keepdims=True)
    xn = x * jax.lax.rsqrt(jnp.maximum(norm2, 1e-24))
    adj = jax.lax.dot_general(
        xn, xn, (((2,), (2,)), ((0,), (0,))),
        preferred_element_type=jnp.float32,
    )

    e = jnp.exp((adj - 1.0) * INV_TEMPERATURE)
    s = jnp.sum(e, axis=-1, keepdims=True)
    rh = 0.5 / s

    t8c = _row_top8_threshold(e)
    t8 = jnp.transpose(t8c, (0, 2, 1))

    thr = jnp.maximum(t8, THRESHOLD * s)
    both = jnp.concatenate((thr, rh), axis=2)
    both_c = jnp.transpose(both, (0, 2, 1))
    thr_c = both_c[:, 0:1, :]
    rh_c = both_c[:, 1:2, :]
    p = jnp.where(e >= thr, e, 0.0) * rh
    q = jnp.where(e >= thr_c, e, 0.0) * rh_c
    o_ref[0] = p + q


def kernel(features):
    B, N, T, D = features.shape
    Tt = 256
    return pl.pallas_call(
        _graph_block_kernel,
        grid=(B, T // Tt),
        in_specs=[pl.BlockSpec((1, N, Tt, D), lambda b, t: (b, 0, t, 0))],
        out_specs=pl.BlockSpec((1, Tt, N, N), lambda b, t: (b, t, 0, 0)),
        out_shape=jax.ShapeDtypeStruct((B, T, N, N), jnp.float32),
        compiler_params=pltpu.CompilerParams(
            dimension_semantics=("parallel", "parallel"),
        ),
    )(features)
